# baseline (speedup 1.0000x reference)
import jax
import jax.numpy as jnp
from jax import lax
from jax.experimental import pallas as pl
from jax.experimental.pallas import tpu as pltpu

_F = 208
_NFIELD = 26
_E = 16
_BB = 128


def _body(x_ref, w_ref, v_ref, fi_ref, out_ref, y_ref):
    x = x_ref[...]
    fi = fi_ref[...]
    v = v_ref[...]

    rows = lax.broadcasted_iota(jnp.int32, (_NFIELD, _F), 0)
    onehot_t = (rows == fi).astype(jnp.float32)
    emb = lax.dot_general(onehot_t, v, (((0,), (0,)), ((), ())),
                          preferred_element_type=jnp.float32)
    emb_t = lax.dot_general(v, onehot_t, (((0,), (0,)), ((), ())),
                            preferred_element_type=jnp.float32)

    s_row = jnp.sum(emb_t, axis=0, keepdims=True)
    q_row = jnp.sum(emb_t * emb_t, axis=0, keepdims=True)

    linear = jnp.sum(x * w_ref[...], axis=1, keepdims=True)
    t = jnp.sum(x * s_row, axis=1, keepdims=True)
    qq = jnp.sum(x * x * q_row, axis=1, keepdims=True)
    inter = 0.5 * (t * t - qq)
    y_ref[...] = jnp.concatenate([linear, inter], axis=1)

    xb = lax.broadcast_in_dim(x, (_BB, _F, _E), (0, 1))
    eb = lax.broadcast_in_dim(emb, (_BB, _F, _E), (1, 2))
    out_ref[...] = xb * eb


def kernel(inputs, w, V, field_index):
    B = inputs.shape[0]
    grid = B // _BB
    new_inputs, y_fm = pl.pallas_call(
        _body,
        grid=(grid,),
        in_specs=[
            pl.BlockSpec((_BB, _F), lambda i: (i, 0)),
            pl.BlockSpec((1, _F), lambda i: (0, 0)),
            pl.BlockSpec((_NFIELD, _E), lambda i: (0, 0)),
            pl.BlockSpec((1, _F), lambda i: (0, 0)),
        ],
        out_specs=[
            pl.BlockSpec((_BB, _F, _E), lambda i: (i, 0, 0)),
            pl.BlockSpec((_BB, 2), lambda i: (i, 0)),
        ],
        out_shape=[
            jax.ShapeDtypeStruct((B, _F, _E), jnp.float32),
            jax.ShapeDtypeStruct((B, 2), jnp.float32),
        ],
    )(inputs, w.reshape(1, _F), V, field_index.reshape(1, _F))
    return (y_fm, new_inputs)


# speedup vs baseline: 3.7513x; 3.7513x over previous
import jax
import jax.numpy as jnp
from jax import lax
from jax.experimental import pallas as pl
from jax.experimental.pallas import tpu as pltpu

_F = 208
_NFIELD = 26
_E = 16
_BB = 512
_NWIN = _F // 8


def _body(x_ref, w_ref, v_ref, fi_ref, out_ref, y_ref, r_ref, sq_ref):
    @pl.when(pl.program_id(0) == 0)
    def _init():
        v = v_ref[...]
        fi = fi_ref[...].astype(jnp.float32)
        lane8 = lax.broadcasted_iota(jnp.int32, (8, 128), 1)
        row8 = lax.broadcasted_iota(jnp.int32, (8, 128), 0)
        sel8 = (lane8 // _E == row8).astype(jnp.float32)
        laneT = lax.broadcasted_iota(jnp.int32, (128, 8), 0)
        rowT = lax.broadcasted_iota(jnp.int32, (128, 8), 1)
        sel8t = (laneT // _E == rowT).astype(jnp.float32)
        vtile = jnp.concatenate([v] * 8, axis=1)
        c_iota = lax.broadcasted_iota(jnp.int32, (_NFIELD, 128), 0)
        for j in range(_NWIN):
            fi_win = fi[:, 8 * j:8 * j + 8]
            fi_rep = lax.dot_general(
                fi_win, sel8, (((1,), (0,)), ((), ())),
                preferred_element_type=jnp.float32,
                precision=lax.Precision.HIGHEST)
            onehot = (c_iota == fi_rep.astype(jnp.int32)).astype(jnp.float32)
            embrow = jnp.sum(onehot * vtile, axis=0, keepdims=True)
            r_ref[pl.ds(8 * j, 8), :] = sel8 * embrow
            s_win = lax.dot_general(
                embrow, sel8t, (((1,), (0,)), ((), ())),
                preferred_element_type=jnp.float32,
                precision=lax.Precision.HIGHEST)
            q_win = lax.dot_general(
                embrow * embrow, sel8t, (((1,), (0,)), ((), ())),
                preferred_element_type=jnp.float32,
                precision=lax.Precision.HIGHEST)
            sq_ref[0:1, pl.ds(8 * j, 8)] = s_win
            sq_ref[1:2, pl.ds(8 * j, 8)] = q_win

    x = x_ref[...]
    for j in range(_NWIN):
        out_ref[:, pl.ds(128 * j, 128)] = lax.dot_general(
            x[:, 8 * j:8 * j + 8], r_ref[pl.ds(8 * j, 8), :],
            (((1,), (0,)), ((), ())),
            preferred_element_type=jnp.float32,
            precision=lax.Precision.HIGHEST)

    s_row = sq_ref[0:1, :]
    q_row = sq_ref[1:2, :]
    linear = jnp.sum(x * w_ref[...], axis=1, keepdims=True)
    t = jnp.sum(x * s_row, axis=1, keepdims=True)
    qq = jnp.sum(x * x * q_row, axis=1, keepdims=True)
    inter = 0.5 * (t * t - qq)
    y_ref[...] = jnp.concatenate([linear, inter], axis=1)


def kernel(inputs, w, V, field_index):
    B = inputs.shape[0]
    grid = B // _BB
    out_flat, y_fm = pl.pallas_call(
        _body,
        grid=(grid,),
        in_specs=[
            pl.BlockSpec((_BB, _F), lambda i: (i, 0)),
            pl.BlockSpec((1, _F), lambda i: (0, 0)),
            pl.BlockSpec((_NFIELD, _E), lambda i: (0, 0)),
            pl.BlockSpec((1, _F), lambda i: (0, 0)),
        ],
        out_specs=[
            pl.BlockSpec((_BB, _F * _E), lambda i: (i, 0)),
            pl.BlockSpec((_BB, 2), lambda i: (i, 0)),
        ],
        out_shape=[
            jax.ShapeDtypeStruct((B, _F * _E), jnp.float32),
            jax.ShapeDtypeStruct((B, 2), jnp.float32),
        ],
        scratch_shapes=[
            pltpu.VMEM((_F, 128), jnp.float32),
            pltpu.VMEM((2, _F), jnp.float32),
        ],
    )(inputs, w.reshape(1, _F), V, field_index.reshape(1, _F))
    return (y_fm, out_flat.reshape(B, _F, _E))


# speedup vs baseline: 4.7983x; 1.2791x over previous
import jax
import jax.numpy as jnp
from jax import lax
from jax.experimental import pallas as pl
from jax.experimental.pallas import tpu as pltpu

_F = 208
_NFIELD = 26
_E = 16
_BB = 512
_W = 16
_NWIN = _F // _W
_NL = _W * _E


def _prep_body(v_ref, fi_ref, rh_ref, rl_ref, sq_ref):
    v = v_ref[...]
    fi = fi_ref[...].astype(jnp.float32)
    lane = lax.broadcasted_iota(jnp.int32, (_W, _NL), 1)
    row = lax.broadcasted_iota(jnp.int32, (_W, _NL), 0)
    selw = (lane // _E == row).astype(jnp.float32)
    laneT = lax.broadcasted_iota(jnp.int32, (_NL, _W), 0)
    rowT = lax.broadcasted_iota(jnp.int32, (_NL, _W), 1)
    selwt = (laneT // _E == rowT).astype(jnp.float32)
    vtile = jnp.concatenate([v] * _W, axis=1)
    c_iota = lax.broadcasted_iota(jnp.int32, (_NFIELD, _NL), 0)
    for j in range(_NWIN):
        fi_win = fi[:, _W * j:_W * j + _W]
        fi_rep = lax.dot_general(
            fi_win, selw, (((1,), (0,)), ((), ())),
            preferred_element_type=jnp.float32,
            precision=lax.Precision.HIGHEST)
        onehot = (c_iota == fi_rep.astype(jnp.int32)).astype(jnp.float32)
        embrow = jnp.sum(onehot * vtile, axis=0, keepdims=True)
        r_full = selw * embrow
        r_hi = r_full.astype(jnp.bfloat16)
        r_lo = (r_full - r_hi.astype(jnp.float32)).astype(jnp.bfloat16)
        rh_ref[pl.ds(_W * j, _W), :] = r_hi
        rl_ref[pl.ds(_W * j, _W), :] = r_lo
        s_win = lax.dot_general(
            embrow, selwt, (((1,), (0,)), ((), ())),
            preferred_element_type=jnp.float32,
            precision=lax.Precision.HIGHEST)
        q_win = lax.dot_general(
            embrow * embrow, selwt, (((1,), (0,)), ((), ())),
            preferred_element_type=jnp.float32,
            precision=lax.Precision.HIGHEST)
        sq_ref[0:1, pl.ds(_W * j, _W)] = s_win
        sq_ref[1:2, pl.ds(_W * j, _W)] = q_win


def _main_body(x_ref, w_ref, rh_ref, rl_ref, sq_ref, out_ref, y_ref):
    x = x_ref[...]
    xh = x.astype(jnp.bfloat16)
    xl = (x - xh.astype(jnp.float32)).astype(jnp.bfloat16)
    dn = (((1,), (0,)), ((), ()))
    for j in range(_NWIN):
        xh_w = xh[:, _W * j:_W * j + _W]
        xl_w = xl[:, _W * j:_W * j + _W]
        rh_w = rh_ref[pl.ds(_W * j, _W), :]
        rl_w = rl_ref[pl.ds(_W * j, _W), :]
        acc = lax.dot_general(xh_w, rh_w, dn,
                              preferred_element_type=jnp.float32)
        acc += lax.dot_general(xh_w, rl_w, dn,
                               preferred_element_type=jnp.float32)
        acc += lax.dot_general(xl_w, rh_w, dn,
                               preferred_element_type=jnp.float32)
        out_ref[:, pl.ds(_NL * j, _NL)] = acc

    s_row = sq_ref[0:1, :]
    q_row = sq_ref[1:2, :]
    linear = jnp.sum(x * w_ref[...], axis=1, keepdims=True)
    t = jnp.sum(x * s_row, axis=1, keepdims=True)
    qq = jnp.sum(x * x * q_row, axis=1, keepdims=True)
    inter = 0.5 * (t * t - qq)
    y_ref[...] = jnp.concatenate([linear, inter], axis=1)


def kernel(inputs, w, V, field_index):
    B = inputs.shape[0]
    r_hi, r_lo, sq = pl.pallas_call(
        _prep_body,
        in_specs=[
            pl.BlockSpec((_NFIELD, _E), lambda: (0, 0)),
            pl.BlockSpec((1, _F), lambda: (0, 0)),
        ],
        out_specs=[
            pl.BlockSpec((_F, _NL), lambda: (0, 0)),
            pl.BlockSpec((_F, _NL), lambda: (0, 0)),
            pl.BlockSpec((2, _F), lambda: (0, 0)),
        ],
        out_shape=[
            jax.ShapeDtypeStruct((_F, _NL), jnp.bfloat16),
            jax.ShapeDtypeStruct((_F, _NL), jnp.bfloat16),
            jax.ShapeDtypeStruct((2, _F), jnp.float32),
        ],
    )(V, field_index.reshape(1, _F))

    grid = B // _BB
    out_flat, y_fm = pl.pallas_call(
        _main_body,
        grid=(grid,),
        in_specs=[
            pl.BlockSpec((_BB, _F), lambda i: (i, 0)),
            pl.BlockSpec((1, _F), lambda i: (0, 0)),
            pl.BlockSpec((_F, _NL), lambda i: (0, 0)),
            pl.BlockSpec((_F, _NL), lambda i: (0, 0)),
            pl.BlockSpec((2, _F), lambda i: (0, 0)),
        ],
        out_specs=[
            pl.BlockSpec((_BB, _F * _E), lambda i: (i, 0)),
            pl.BlockSpec((_BB, 2), lambda i: (i, 0)),
        ],
        out_shape=[
            jax.ShapeDtypeStruct((B, _F * _E), jnp.float32),
            jax.ShapeDtypeStruct((B, 2), jnp.float32),
        ],
    )(inputs, w.reshape(1, _F), r_hi, r_lo, sq)
    return (y_fm, out_flat.reshape(B, _F, _E))


# speedup vs baseline: 11.6540x; 2.4288x over previous
import jax
import jax.numpy as jnp
from jax import lax
from jax.experimental import pallas as pl
from jax.experimental.pallas import tpu as pltpu
from jax.experimental.pallas import tpu_sc as plsc

_F = 208
_NFIELD = 26
_E = 16
_FE = _F * _E
_LB = 1024
_NC = 2
_NS = 16
_NW = _NC * _NS
_CL = 256


def _prep_body(v_ref, fi_ref, emb_ref, sq_ref):
    v = v_ref[...]
    fi = fi_ref[...]
    rows = lax.broadcasted_iota(jnp.int32, (_NFIELD, _F), 0)
    onehot_t = (rows == fi).astype(jnp.float32)
    emb = lax.dot_general(onehot_t, v, (((0,), (0,)), ((), ())),
                          preferred_element_type=jnp.float32,
                          precision=lax.Precision.HIGHEST)
    sq_ref[:, 0:1] = jnp.sum(emb, axis=1, keepdims=True)
    sq_ref[:, 1:2] = jnp.sum(emb * emb, axis=1, keepdims=True)
    e_iota = lax.broadcasted_iota(jnp.int32, (_E, _E * _E), 0)
    l_iota = lax.broadcasted_iota(jnp.int32, (_E, _E * _E), 1)
    sel16 = (l_iota // _E == e_iota).astype(jnp.bfloat16)
    emb_hi = emb.astype(jnp.bfloat16)
    emb_lo = (emb - emb_hi.astype(jnp.float32)).astype(jnp.bfloat16)
    dn = (((1,), (0,)), ((), ()))
    splat = lax.dot_general(emb_hi, sel16, dn,
                            preferred_element_type=jnp.float32)
    splat += lax.dot_general(emb_lo, sel16, dn,
                             preferred_element_type=jnp.float32)
    emb_ref[...] = splat


def _y_body(x_ref, w_ref, sq_ref, y_ref):
    xt = x_ref[...]
    wcol = w_ref[...]
    scol = sq_ref[:, 0:1]
    qcol = sq_ref[:, 1:2]
    lin = jnp.sum(xt * wcol, axis=0, keepdims=True)
    t = jnp.sum(xt * scol, axis=0, keepdims=True)
    qq = jnp.sum(xt * xt * qcol, axis=0, keepdims=True)
    inter = 0.5 * (t * t - qq)
    y_ref[...] = jnp.concatenate([lin, inter], axis=0)


def _sc_body(x_hbm, emb_hbm, out_hbm, x_v, emb_v, ob0, ob1, sem0, sem1):
    wid = lax.axis_index("s") * _NC + lax.axis_index("c")
    base = wid * (16384 // _NW)
    pltpu.sync_copy(emb_hbm, emb_v)

    for c in range(16384 // _NW // _CL):
        cb = base + c * _CL
        pltpu.sync_copy(x_hbm.at[:, pl.ds(cb, _CL)], x_v)

        def fill(buf_ref, f):
            xv = [x_v[f, pl.ds(16 * i, 16)] for i in range(_CL // 16)]
            for e in range(_E):
                s_vec = emb_v[f, pl.ds(16 * e, 16)]
                for i in range(_CL // 16):
                    buf_ref[e, pl.ds(16 * i, 16)] = xv[i] * s_vec

        def body(k, carry):
            f0 = 2 * k
            f1 = 2 * k + 1

            @pl.when(k > 0)
            def _wait0():
                pltpu.make_async_copy(
                    out_hbm.at[pl.ds(0, _E), pl.ds(0, _CL)], ob0, sem0).wait()

            fill(ob0, f0)
            pltpu.async_copy(
                ob0, out_hbm.at[pl.ds(16 * f0, _E), pl.ds(cb, _CL)], sem0)

            @pl.when(k > 0)
            def _wait1():
                pltpu.make_async_copy(
                    out_hbm.at[pl.ds(0, _E), pl.ds(0, _CL)], ob1, sem1).wait()

            fill(ob1, f1)
            pltpu.async_copy(
                ob1, out_hbm.at[pl.ds(16 * f1, _E), pl.ds(cb, _CL)], sem1)
            return carry

        lax.fori_loop(0, _F // 2, body, 0)
        pltpu.make_async_copy(
            out_hbm.at[pl.ds(0, _E), pl.ds(0, _CL)], ob0, sem0).wait()
        pltpu.make_async_copy(
            out_hbm.at[pl.ds(0, _E), pl.ds(0, _CL)], ob1, sem1).wait()


def kernel(inputs, w, V, field_index):
    B = inputs.shape[0]
    xt = inputs.T
    emb_flat, sq = pl.pallas_call(
        _prep_body,
        in_specs=[
            pl.BlockSpec((_NFIELD, _E), lambda: (0, 0)),
            pl.BlockSpec((1, _F), lambda: (0, 0)),
        ],
        out_specs=[
            pl.BlockSpec((_F, _E * _E), lambda: (0, 0)),
            pl.BlockSpec((_F, 2), lambda: (0, 0)),
        ],
        out_shape=[
            jax.ShapeDtypeStruct((_F, _E * _E), jnp.float32),
            jax.ShapeDtypeStruct((_F, 2), jnp.float32),
        ],
    )(V, field_index.reshape(1, _F))

    grid = B // _LB
    y_t = pl.pallas_call(
        _y_body,
        grid=(grid,),
        in_specs=[
            pl.BlockSpec((_F, _LB), lambda i: (0, i)),
            pl.BlockSpec((_F, 1), lambda i: (0, 0)),
            pl.BlockSpec((_F, 2), lambda i: (0, 0)),
        ],
        out_specs=pl.BlockSpec((2, _LB), lambda i: (0, i)),
        out_shape=jax.ShapeDtypeStruct((2, B), jnp.float32),
    )(xt, w.reshape(_F, 1), sq)

    mesh = plsc.VectorSubcoreMesh(core_axis_name="c", subcore_axis_name="s")
    sc_main = pl.kernel(
        _sc_body,
        mesh=mesh,
        out_type=jax.ShapeDtypeStruct((_FE, B), jnp.float32),
        scratch_types=[
            pltpu.VMEM((_F, _CL), jnp.float32),
            pltpu.VMEM((_F, _E * _E), jnp.float32),
            pltpu.VMEM((_E, _CL), jnp.float32),
            pltpu.VMEM((_E, _CL), jnp.float32),
            pltpu.SemaphoreType.DMA,
            pltpu.SemaphoreType.DMA,
        ],
    )
    out_t = sc_main(xt, emb_flat)
    return (y_t.T, out_t.T.reshape(B, _F, _E))


# speedup vs baseline: 19.1248x; 1.6410x over previous
import jax
import jax.numpy as jnp
from jax import lax
from jax.experimental import pallas as pl
from jax.experimental.pallas import tpu as pltpu

_F = 208
_NFIELD = 26
_E = 16
_FE = _F * _E
_LB = 1024


def _prep_body(v_ref, fi_ref, emb_ref, sq_ref):
    v = v_ref[...]
    fi = fi_ref[...]
    rows = lax.broadcasted_iota(jnp.int32, (_NFIELD, _F), 0)
    onehot_t = (rows == fi).astype(jnp.float32)
    emb = lax.dot_general(onehot_t, v, (((0,), (0,)), ((), ())),
                          preferred_element_type=jnp.float32,
                          precision=lax.Precision.HIGHEST)
    sq_ref[:, 0:1] = jnp.sum(emb, axis=1, keepdims=True)
    sq_ref[:, 1:2] = jnp.sum(emb * emb, axis=1, keepdims=True)
    r_iota = lax.broadcasted_iota(jnp.int32, (_FE, _F), 0)
    f_iota = lax.broadcasted_iota(jnp.int32, (_FE, _F), 1)
    sel = (r_iota // _E == f_iota).astype(jnp.bfloat16)
    emb_hi = emb.astype(jnp.bfloat16)
    emb_lo = (emb - emb_hi.astype(jnp.float32)).astype(jnp.bfloat16)
    dn = (((1,), (0,)), ((), ()))
    emb_rep = lax.dot_general(sel, emb_hi, dn,
                              preferred_element_type=jnp.float32)
    emb_rep += lax.dot_general(sel, emb_lo, dn,
                               preferred_element_type=jnp.float32)
    re_iota = lax.broadcasted_iota(jnp.int32, (_FE, _E), 0)
    e_iota = lax.broadcasted_iota(jnp.int32, (_FE, _E), 1)
    pick = (re_iota % _E == e_iota).astype(jnp.float32)
    emb_ref[...] = jnp.sum(emb_rep * pick, axis=1, keepdims=True)


def _main_body(x_ref, w_ref, emb_ref, sq_ref, out_ref, y_ref):
    xt = x_ref[...]
    x3 = lax.broadcast_in_dim(xt, (_F, _E, _LB), (0, 2))
    xrep = x3.reshape(_FE, _LB)
    out_ref[...] = xrep * emb_ref[...]

    wcol = w_ref[...]
    scol = sq_ref[:, 0:1]
    qcol = sq_ref[:, 1:2]
    lin = jnp.sum(xt * wcol, axis=0, keepdims=True)
    t = jnp.sum(xt * scol, axis=0, keepdims=True)
    qq = jnp.sum(xt * xt * qcol, axis=0, keepdims=True)
    inter = 0.5 * (t * t - qq)
    y_ref[...] = jnp.concatenate([lin, inter], axis=0)


def kernel(inputs, w, V, field_index):
    B = inputs.shape[0]
    emb_flat, sq = pl.pallas_call(
        _prep_body,
        in_specs=[
            pl.BlockSpec((_NFIELD, _E), lambda: (0, 0)),
            pl.BlockSpec((1, _F), lambda: (0, 0)),
        ],
        out_specs=[
            pl.BlockSpec((_FE, 1), lambda: (0, 0)),
            pl.BlockSpec((_F, 2), lambda: (0, 0)),
        ],
        out_shape=[
            jax.ShapeDtypeStruct((_FE, 1), jnp.float32),
            jax.ShapeDtypeStruct((_F, 2), jnp.float32),
        ],
    )(V, field_index.reshape(1, _F))

    grid = B // _LB
    out_t, y_t = pl.pallas_call(
        _main_body,
        grid=(grid,),
        in_specs=[
            pl.BlockSpec((_F, _LB), lambda i: (0, i)),
            pl.BlockSpec((_F, 1), lambda i: (0, 0)),
            pl.BlockSpec((_FE, 1), lambda i: (0, 0)),
            pl.BlockSpec((_F, 2), lambda i: (0, 0)),
        ],
        out_specs=[
            pl.BlockSpec((_FE, _LB), lambda i: (0, i)),
            pl.BlockSpec((2, _LB), lambda i: (0, i)),
        ],
        out_shape=[
            jax.ShapeDtypeStruct((_FE, B), jnp.float32),
            jax.ShapeDtypeStruct((2, B), jnp.float32),
        ],
    )(inputs.T, w.reshape(_F, 1), emb_flat, sq)
    return (y_t.T, out_t.T.reshape(B, _F, _E))
